# baseline (speedup 1.0000x reference)
import functools

import jax
import jax.numpy as jnp
from jax import lax
from jax.experimental import pallas as pl
from jax.experimental.pallas import tpu as pltpu
from jax.experimental.pallas import tpu_sc as plsc

EPS_ = 1e-4
BLK = 2048
NEXP = 64
LANES = 16
NWORK = 32


def _ln_logits(x, w, b, gt):
    mean = jnp.mean(x, axis=1, keepdims=True)
    xc = x - mean
    var = jnp.mean(xc * xc, axis=1, keepdims=True)
    hn = xc * lax.rsqrt(var + 1e-5) * w + b
    hn = jnp.clip(hn, -50.0, 50.0)
    logits = jax.lax.dot_general(
        hn, gt, (((1,), (0,)), ((), ())),
        preferred_element_type=jnp.float32,
    )
    return jnp.clip(logits, -10.0, 10.0)


def _tc_logits_kernel(x_ref, w_ref, b_ref, gt_ref, logits_ref):
    logits_ref[...] = _ln_logits(x_ref[...], w_ref[...], b_ref[...], gt_ref[...])


def _tc_full_kernel(x_ref, w_ref, b_ref, gt_ref, logits_ref, probs_ref, idx_ref):
    logits = _ln_logits(x_ref[...], w_ref[...], b_ref[...], gt_ref[...])
    logits_ref[...] = logits
    m = jnp.max(logits, axis=1, keepdims=True)
    e = jnp.exp(logits - m)
    z = jnp.sum(e, axis=1, keepdims=True)
    p = jnp.clip(e / z, EPS_, 1.0)
    iota = lax.broadcasted_iota(jnp.int32, p.shape, 1)
    m1 = jnp.max(p, axis=1, keepdims=True)
    i1 = jnp.min(jnp.where(p == m1, iota, NEXP), axis=1, keepdims=True)
    pm = jnp.where(iota == i1, -1.0, p)
    m2 = jnp.max(pm, axis=1, keepdims=True)
    i2 = jnp.min(jnp.where(pm == m2, iota, NEXP), axis=1, keepdims=True)
    s = jnp.maximum(m1 + m2, EPS_)
    probs_ref[...] = jnp.concatenate([m1 / s, m2 / s], axis=1)
    idx_ref[...] = jnp.concatenate([i1, i2], axis=1)


def _sc_topk_body(logits_hbm, probs_hbm, idx_hbm, buf_v, probs_v, idx_v):
    t_per_w = buf_v.shape[0] // NEXP
    wid = lax.axis_index("s") * 2 + lax.axis_index("c")
    base = wid * (t_per_w * NEXP)
    pltpu.sync_copy(logits_hbm.at[pl.ds(base, t_per_w * NEXP)], buf_v)

    lane = lax.iota(jnp.int32, LANES)
    n_groups = t_per_w // LANES

    def key_decode(k):
        kb = k ^ (lax.shift_right_arithmetic(k, 31) & jnp.int32(0x7FFFFFFF))
        return lax.bitcast_convert_type(kb, jnp.float32)

    def group(g, _):
        idx_base = g * (LANES * NEXP) + lane * NEXP
        m1 = jnp.full((LANES,), jnp.int32(-0x80000000))
        m2 = jnp.full((LANES,), jnp.int32(-0x80000000))
        z = jnp.zeros((LANES,), jnp.float32)
        for e in range(NEXP):
            l = plsc.load_gather(buf_v, [idx_base + e])
            z = z + jnp.exp(l)
            b = lax.bitcast_convert_type(l, jnp.int32)
            k = b ^ (lax.shift_right_arithmetic(b, 31) & jnp.int32(0x7FFFFFFF))
            k = (k & jnp.int32(~0x3F)) | jnp.int32(NEXP - 1 - e)
            m2 = jnp.maximum(m2, jnp.minimum(m1, k))
            m1 = jnp.maximum(m1, k)
        i1 = jnp.int32(NEXP - 1) - (m1 & jnp.int32(0x3F))
        i2 = jnp.int32(NEXP - 1) - (m2 & jnp.int32(0x3F))
        v1 = key_decode(m1 & jnp.int32(~0x3F))
        v2 = key_decode(m2 & jnp.int32(~0x3F))
        p1 = jnp.clip(jnp.exp(v1) / z, EPS_, 1.0)
        p2 = jnp.clip(jnp.exp(v2) / z, EPS_, 1.0)
        s = jnp.maximum(p1 + p2, EPS_)
        o1 = p1 / s
        o2 = p2 / s
        pidx = g * (LANES * 2) + lane * 2
        plsc.store_scatter(probs_v, [pidx], o1)
        plsc.store_scatter(probs_v, [pidx + 1], o2)
        plsc.store_scatter(idx_v, [pidx], i1)
        plsc.store_scatter(idx_v, [pidx + 1], i2)
        return _

    lax.fori_loop(0, n_groups, group, 0, unroll=2)
    out_base = wid * (t_per_w * 2)
    pltpu.sync_copy(probs_v, probs_hbm.at[pl.ds(out_base, t_per_w * 2)])
    pltpu.sync_copy(idx_v, idx_hbm.at[pl.ds(out_base, t_per_w * 2)])


@jax.jit
def kernel(hidden_states, ln_weight, ln_bias, gate_weight):
    B, S, D = hidden_states.shape
    N = B * S
    x = hidden_states.reshape(N, D)
    w = ln_weight.reshape(1, D)
    b = ln_bias.reshape(1, D)
    gt = gate_weight.T
    E = gate_weight.shape[0]
    half = N // 2

    common = dict(
        in_specs=[
            pl.BlockSpec((BLK, D), lambda i: (i, 0)),
            pl.BlockSpec((1, D), lambda i: (0, 0)),
            pl.BlockSpec((1, D), lambda i: (0, 0)),
            pl.BlockSpec((D, E), lambda i: (0, 0)),
        ],
        grid=(half // BLK,),
    )
    logits_a = pl.pallas_call(
        _tc_logits_kernel,
        out_specs=pl.BlockSpec((BLK, E), lambda i: (i, 0)),
        out_shape=jax.ShapeDtypeStruct((half, E), jnp.float32),
        **common,
    )(x[:half], w, b, gt)

    t_per_w = half // NWORK
    mesh = plsc.VectorSubcoreMesh(core_axis_name="c", subcore_axis_name="s")
    sc_topk = functools.partial(
        pl.kernel,
        mesh=mesh,
        compiler_params=pltpu.CompilerParams(needs_layout_passes=False),
        out_type=[
            jax.ShapeDtypeStruct((half * 2,), jnp.float32),
            jax.ShapeDtypeStruct((half * 2,), jnp.int32),
        ],
        scratch_types=[
            pltpu.VMEM((t_per_w * NEXP,), jnp.float32),
            pltpu.VMEM((t_per_w * 2,), jnp.float32),
            pltpu.VMEM((t_per_w * 2,), jnp.int32),
        ],
    )(_sc_topk_body)
    probs_a, idx_a = sc_topk(logits_a.reshape(-1))

    logits_b, probs_b, idx_b = pl.pallas_call(
        _tc_full_kernel,
        out_specs=[
            pl.BlockSpec((BLK, E), lambda i: (i, 0)),
            pl.BlockSpec((BLK, 2), lambda i: (i, 0)),
            pl.BlockSpec((BLK, 2), lambda i: (i, 0)),
        ],
        out_shape=[
            jax.ShapeDtypeStruct((half, E), jnp.float32),
            jax.ShapeDtypeStruct((half, 2), jnp.float32),
            jax.ShapeDtypeStruct((half, 2), jnp.int32),
        ],
        **common,
    )(x[half:], w, b, gt)

    probs = jnp.concatenate([probs_a.reshape(half, 2), probs_b], axis=0)
    idx = jnp.concatenate([idx_a.reshape(half, 2), idx_b], axis=0)
    logits = jnp.concatenate([logits_a, logits_b], axis=0)
    return probs, idx, logits


# speedup vs baseline: 1.6887x; 1.6887x over previous
import functools

import jax
import jax.numpy as jnp
from jax import lax
from jax.experimental import pallas as pl
from jax.experimental.pallas import tpu as pltpu
from jax.experimental.pallas import tpu_sc as plsc

EPS_ = 1e-4
BLK = 2048
NEXP = 64
LANES = 16
NWORK = 32


def _ln_logits(x, w, b, gt):
    mean = jnp.mean(x, axis=1, keepdims=True)
    xc = x - mean
    var = jnp.mean(xc * xc, axis=1, keepdims=True)
    hn = xc * lax.rsqrt(var + 1e-5) * w + b
    hn = jnp.clip(hn, -50.0, 50.0)
    logits = jax.lax.dot_general(
        hn, gt, (((1,), (0,)), ((), ())),
        preferred_element_type=jnp.float32,
    )
    return jnp.clip(logits, -10.0, 10.0)


def _tc_logits_kernel(x_ref, w_ref, b_ref, gt_ref, logits_ref):
    logits_ref[...] = _ln_logits(x_ref[...], w_ref[...], b_ref[...], gt_ref[...])


def _tc_full_kernel(x_ref, w_ref, b_ref, gt_ref, logits_ref, probs_ref, idx_ref):
    logits = _ln_logits(x_ref[...], w_ref[...], b_ref[...], gt_ref[...])
    logits_ref[...] = logits
    m = jnp.max(logits, axis=1, keepdims=True)
    e = jnp.exp(logits - m)
    z = jnp.sum(e, axis=1, keepdims=True)
    p = jnp.clip(e / z, EPS_, 1.0)
    iota = lax.broadcasted_iota(jnp.int32, p.shape, 1)
    m1 = jnp.max(p, axis=1, keepdims=True)
    i1 = jnp.min(jnp.where(p == m1, iota, NEXP), axis=1, keepdims=True)
    pm = jnp.where(iota == i1, -1.0, p)
    m2 = jnp.max(pm, axis=1, keepdims=True)
    i2 = jnp.min(jnp.where(pm == m2, iota, NEXP), axis=1, keepdims=True)
    s = jnp.maximum(m1 + m2, EPS_)
    probs_ref[...] = jnp.concatenate([m1 / s, m2 / s], axis=1)
    idx_ref[...] = jnp.concatenate([i1, i2], axis=1)


def _sc_topk_body(logits_hbm, probs_hbm, idx_hbm, buf_v, probs_v, idx_v):
    t_per_w = buf_v.shape[0] // NEXP
    wid = lax.axis_index("s") * 2 + lax.axis_index("c")
    base = wid * (t_per_w * NEXP)
    pltpu.sync_copy(logits_hbm.at[pl.ds(base, t_per_w * NEXP)], buf_v)

    lane = lax.iota(jnp.int32, LANES)
    n_groups = t_per_w // LANES

    def key_decode(k):
        kb = k ^ (lax.shift_right_arithmetic(k, 31) & jnp.int32(0x7FFFFFFF))
        return lax.bitcast_convert_type(kb, jnp.float32)

    def group(g, _):
        idx_base = g * (LANES * NEXP) + lane * NEXP
        m1 = jnp.full((LANES,), jnp.int32(-0x80000000))
        m2 = jnp.full((LANES,), jnp.int32(-0x80000000))
        z = jnp.zeros((LANES,), jnp.float32)
        for e in range(NEXP):
            l = plsc.load_gather(buf_v, [idx_base + e])
            z = z + jnp.exp(l)
            b = lax.bitcast_convert_type(l, jnp.int32)
            k = b ^ (lax.shift_right_arithmetic(b, 31) & jnp.int32(0x7FFFFFFF))
            k = (k & jnp.int32(~0x3F)) | jnp.int32(NEXP - 1 - e)
            m2 = jnp.maximum(m2, jnp.minimum(m1, k))
            m1 = jnp.maximum(m1, k)
        i1 = jnp.int32(NEXP - 1) - (m1 & jnp.int32(0x3F))
        i2 = jnp.int32(NEXP - 1) - (m2 & jnp.int32(0x3F))
        v1 = key_decode(m1 & jnp.int32(~0x3F))
        v2 = key_decode(m2 & jnp.int32(~0x3F))
        p1 = jnp.clip(jnp.exp(v1) / z, EPS_, 1.0)
        p2 = jnp.clip(jnp.exp(v2) / z, EPS_, 1.0)
        s = jnp.maximum(p1 + p2, EPS_)
        o1 = p1 / s
        o2 = p2 / s
        pidx = g * (LANES * 2) + lane * 2
        plsc.store_scatter(probs_v, [pidx], o1)
        plsc.store_scatter(probs_v, [pidx + 1], o2)
        plsc.store_scatter(idx_v, [pidx], i1)
        plsc.store_scatter(idx_v, [pidx + 1], i2)
        return _

    lax.fori_loop(0, n_groups, group, 0, unroll=2)
    out_base = wid * (t_per_w * 2)
    pltpu.sync_copy(probs_v, probs_hbm.at[pl.ds(out_base, t_per_w * 2)])
    pltpu.sync_copy(idx_v, idx_hbm.at[pl.ds(out_base, t_per_w * 2)])


@jax.jit
def kernel(hidden_states, ln_weight, ln_bias, gate_weight):
    B, S, D = hidden_states.shape
    N = B * S
    x = hidden_states.reshape(N, D)
    w = ln_weight.reshape(1, D)
    b = ln_bias.reshape(1, D)
    gt = gate_weight.T
    E = gate_weight.shape[0]
    half = N // 2

    nblk_half = half // BLK

    def specs_for(off):
        return [
            pl.BlockSpec((BLK, D), lambda i: (i + off, 0)),
            pl.BlockSpec((1, D), lambda i: (0, 0)),
            pl.BlockSpec((1, D), lambda i: (0, 0)),
            pl.BlockSpec((D, E), lambda i: (0, 0)),
        ]

    logits_a = pl.pallas_call(
        _tc_logits_kernel,
        grid=(nblk_half,),
        in_specs=specs_for(0),
        out_specs=pl.BlockSpec((BLK, E), lambda i: (i, 0)),
        out_shape=jax.ShapeDtypeStruct((half, E), jnp.float32),
    )(x, w, b, gt)

    t_per_w = half // NWORK
    mesh = plsc.VectorSubcoreMesh(core_axis_name="c", subcore_axis_name="s")
    sc_topk = functools.partial(
        pl.kernel,
        mesh=mesh,
        compiler_params=pltpu.CompilerParams(needs_layout_passes=False),
        out_type=[
            jax.ShapeDtypeStruct((half * 2,), jnp.float32),
            jax.ShapeDtypeStruct((half * 2,), jnp.int32),
        ],
        scratch_types=[
            pltpu.VMEM((t_per_w * NEXP,), jnp.float32),
            pltpu.VMEM((t_per_w * 2,), jnp.float32),
            pltpu.VMEM((t_per_w * 2,), jnp.int32),
        ],
    )(_sc_topk_body)
    probs_a, idx_a = sc_topk(logits_a.reshape(-1))

    logits_b, probs_b, idx_b = pl.pallas_call(
        _tc_full_kernel,
        grid=(nblk_half,),
        in_specs=specs_for(nblk_half),
        out_specs=[
            pl.BlockSpec((BLK, E), lambda i: (i, 0)),
            pl.BlockSpec((BLK, 2), lambda i: (i, 0)),
            pl.BlockSpec((BLK, 2), lambda i: (i, 0)),
        ],
        out_shape=[
            jax.ShapeDtypeStruct((half, E), jnp.float32),
            jax.ShapeDtypeStruct((half, 2), jnp.float32),
            jax.ShapeDtypeStruct((half, 2), jnp.int32),
        ],
    )(x, w, b, gt)

    probs = jnp.concatenate([probs_a.reshape(half, 2), probs_b], axis=0)
    idx = jnp.concatenate([idx_a.reshape(half, 2), idx_b], axis=0)
    logits = jnp.concatenate([logits_a, logits_b], axis=0)
    return probs, idx, logits


# speedup vs baseline: 1.7171x; 1.0169x over previous
import functools

import jax
import jax.numpy as jnp
from jax import lax
from jax.experimental import pallas as pl
from jax.experimental.pallas import tpu as pltpu
from jax.experimental.pallas import tpu_sc as plsc

EPS_ = 1e-4
BLK = 2048
NEXP = 64
LANES = 16
NWORK = 32


def _ln_logits(x, w, b, gt):
    mean = jnp.mean(x, axis=1, keepdims=True)
    xc = x - mean
    var = jnp.mean(xc * xc, axis=1, keepdims=True)
    hn = xc * lax.rsqrt(var + 1e-5) * w + b
    hn = jnp.clip(hn, -50.0, 50.0)
    logits = jax.lax.dot_general(
        hn, gt, (((1,), (0,)), ((), ())),
        preferred_element_type=jnp.float32,
    )
    return jnp.clip(logits, -10.0, 10.0)


def _tc_logits_kernel(x_ref, w_ref, b_ref, gt_ref, logits_ref):
    logits_ref[...] = _ln_logits(x_ref[...], w_ref[...], b_ref[...], gt_ref[...])


def _tc_full_kernel(x_ref, w_ref, b_ref, gt_ref, logits_ref, probs_ref, idx_ref):
    logits = _ln_logits(x_ref[...], w_ref[...], b_ref[...], gt_ref[...])
    logits_ref[...] = logits
    m = jnp.max(logits, axis=1, keepdims=True)
    e = jnp.exp(logits - m)
    z = jnp.sum(e, axis=1, keepdims=True)
    p = jnp.clip(e / z, EPS_, 1.0)
    iota = lax.broadcasted_iota(jnp.int32, p.shape, 1)
    m1 = jnp.max(p, axis=1, keepdims=True)
    i1 = jnp.min(jnp.where(p == m1, iota, NEXP), axis=1, keepdims=True)
    pm = jnp.where(iota == i1, -1.0, p)
    m2 = jnp.max(pm, axis=1, keepdims=True)
    i2 = jnp.min(jnp.where(pm == m2, iota, NEXP), axis=1, keepdims=True)
    s = jnp.maximum(m1 + m2, EPS_)
    probs_ref[...] = jnp.concatenate([m1 / s, m2 / s], axis=1)
    idx_ref[...] = jnp.concatenate([i1, i2], axis=1)


def _sc_topk_body(logits_hbm, probs_hbm, idx_hbm, buf_v, probs_v, idx_v):
    t_per_w = buf_v.shape[0] // NEXP
    wid = lax.axis_index("s") * 2 + lax.axis_index("c")
    base = wid * (t_per_w * NEXP)
    pltpu.sync_copy(logits_hbm.at[pl.ds(base, t_per_w * NEXP)], buf_v)

    lane = lax.iota(jnp.int32, LANES)
    n_groups = t_per_w // LANES

    def key_decode(k):
        kb = k ^ (lax.shift_right_arithmetic(k, 31) & jnp.int32(0x7FFFFFFF))
        return lax.bitcast_convert_type(kb, jnp.float32)

    def group(g, _):
        idx_base = g * (LANES * NEXP) + lane * NEXP
        m1 = jnp.full((LANES,), jnp.int32(-0x80000000))
        m2 = jnp.full((LANES,), jnp.int32(-0x80000000))
        z = jnp.zeros((LANES,), jnp.float32)
        for e in range(NEXP):
            l = plsc.load_gather(buf_v, [idx_base + e])
            z = z + jnp.exp(l)
            b = lax.bitcast_convert_type(l, jnp.int32)
            k = b ^ (lax.shift_right_arithmetic(b, 31) & jnp.int32(0x7FFFFFFF))
            k = (k & jnp.int32(~0x3F)) | jnp.int32(NEXP - 1 - e)
            m2 = jnp.maximum(m2, jnp.minimum(m1, k))
            m1 = jnp.maximum(m1, k)
        i1 = jnp.int32(NEXP - 1) - (m1 & jnp.int32(0x3F))
        i2 = jnp.int32(NEXP - 1) - (m2 & jnp.int32(0x3F))
        v1 = key_decode(m1 & jnp.int32(~0x3F))
        v2 = key_decode(m2 & jnp.int32(~0x3F))
        p1 = jnp.clip(jnp.exp(v1) / z, EPS_, 1.0)
        p2 = jnp.clip(jnp.exp(v2) / z, EPS_, 1.0)
        s = jnp.maximum(p1 + p2, EPS_)
        o1 = p1 / s
        o2 = p2 / s
        pidx = g * (LANES * 2) + lane * 2
        plsc.store_scatter(probs_v, [pidx], o1)
        plsc.store_scatter(probs_v, [pidx + 1], o2)
        plsc.store_scatter(idx_v, [pidx], i1)
        plsc.store_scatter(idx_v, [pidx + 1], i2)
        return _

    lax.fori_loop(0, n_groups, group, 0, unroll=2)
    out_base = wid * (t_per_w * 2)
    pltpu.sync_copy(probs_v, probs_hbm.at[pl.ds(out_base, t_per_w * 2)])
    pltpu.sync_copy(idx_v, idx_hbm.at[pl.ds(out_base, t_per_w * 2)])


@jax.jit
def kernel(hidden_states, ln_weight, ln_bias, gate_weight):
    B, S, D = hidden_states.shape
    N = B * S
    x = hidden_states.reshape(N, D)
    w = ln_weight.reshape(1, D)
    b = ln_bias.reshape(1, D)
    gt = gate_weight.T
    E = gate_weight.shape[0]
    na = N // 4
    nb = N - na
    nblk_a = na // BLK
    nblk_b = nb // BLK

    def specs_for(off):
        return [
            pl.BlockSpec((BLK, D), lambda i: (i + off, 0)),
            pl.BlockSpec((1, D), lambda i: (0, 0)),
            pl.BlockSpec((1, D), lambda i: (0, 0)),
            pl.BlockSpec((D, E), lambda i: (0, 0)),
        ]

    logits_a = pl.pallas_call(
        _tc_logits_kernel,
        grid=(nblk_a,),
        in_specs=specs_for(0),
        out_specs=pl.BlockSpec((BLK, E), lambda i: (i, 0)),
        out_shape=jax.ShapeDtypeStruct((na, E), jnp.float32),
    )(x, w, b, gt)

    t_per_w = na // NWORK
    mesh = plsc.VectorSubcoreMesh(core_axis_name="c", subcore_axis_name="s")
    sc_topk = functools.partial(
        pl.kernel,
        mesh=mesh,
        compiler_params=pltpu.CompilerParams(needs_layout_passes=False),
        out_type=[
            jax.ShapeDtypeStruct((na * 2,), jnp.float32),
            jax.ShapeDtypeStruct((na * 2,), jnp.int32),
        ],
        scratch_types=[
            pltpu.VMEM((t_per_w * NEXP,), jnp.float32),
            pltpu.VMEM((t_per_w * 2,), jnp.float32),
            pltpu.VMEM((t_per_w * 2,), jnp.int32),
        ],
    )(_sc_topk_body)
    probs_a, idx_a = sc_topk(logits_a.reshape(-1))

    logits_b, probs_b, idx_b = pl.pallas_call(
        _tc_full_kernel,
        grid=(nblk_b,),
        in_specs=specs_for(nblk_a),
        out_specs=[
            pl.BlockSpec((BLK, E), lambda i: (i, 0)),
            pl.BlockSpec((BLK, 2), lambda i: (i, 0)),
            pl.BlockSpec((BLK, 2), lambda i: (i, 0)),
        ],
        out_shape=[
            jax.ShapeDtypeStruct((nb, E), jnp.float32),
            jax.ShapeDtypeStruct((nb, 2), jnp.float32),
            jax.ShapeDtypeStruct((nb, 2), jnp.int32),
        ],
    )(x, w, b, gt)

    probs = jnp.concatenate([probs_a.reshape(na, 2), probs_b], axis=0)
    idx = jnp.concatenate([idx_a.reshape(na, 2), idx_b], axis=0)
    logits = jnp.concatenate([logits_a, logits_b], axis=0)
    return probs, idx, logits


# speedup vs baseline: 1.7186x; 1.0008x over previous
import functools

import jax
import jax.numpy as jnp
from jax import lax
from jax.experimental import pallas as pl
from jax.experimental.pallas import tpu as pltpu
from jax.experimental.pallas import tpu_sc as plsc

EPS_ = 1e-4
BLK = 2048
NEXP = 64
LANES = 16
NWORK = 32


def _ln_logits(x, w, b, gt):
    mean = jnp.mean(x, axis=1, keepdims=True)
    xc = x - mean
    var = jnp.mean(xc * xc, axis=1, keepdims=True)
    hn = xc * lax.rsqrt(var + 1e-5) * w + b
    hn = jnp.clip(hn, -50.0, 50.0)
    logits = jax.lax.dot_general(
        hn, gt, (((1,), (0,)), ((), ())),
        preferred_element_type=jnp.float32,
    )
    return jnp.clip(logits, -10.0, 10.0)


def _tc_logits_kernel(x_ref, w_ref, b_ref, gt_ref, logits_ref):
    logits_ref[...] = _ln_logits(x_ref[...], w_ref[...], b_ref[...], gt_ref[...])


def _tc_full_kernel(x_ref, w_ref, b_ref, gt_ref, logits_ref, probs_ref, idx_ref):
    logits = _ln_logits(x_ref[...], w_ref[...], b_ref[...], gt_ref[...])
    logits_ref[...] = logits
    m = jnp.max(logits, axis=1, keepdims=True)
    e = jnp.exp(logits - m)
    z = jnp.sum(e, axis=1, keepdims=True)
    p = jnp.clip(e / z, EPS_, 1.0)
    iota = lax.broadcasted_iota(jnp.int32, p.shape, 1)
    m1 = jnp.max(p, axis=1, keepdims=True)
    i1 = jnp.min(jnp.where(p == m1, iota, NEXP), axis=1, keepdims=True)
    pm = jnp.where(iota == i1, -1.0, p)
    m2 = jnp.max(pm, axis=1, keepdims=True)
    i2 = jnp.min(jnp.where(pm == m2, iota, NEXP), axis=1, keepdims=True)
    s = jnp.maximum(m1 + m2, EPS_)
    probs_ref[...] = jnp.concatenate([m1 / s, m2 / s], axis=1)
    idx_ref[...] = jnp.concatenate([i1, i2], axis=1)


def _sc_topk_body(logits_hbm, probs_hbm, idx_hbm, buf_v, probs_v, idx_v):
    t_per_w = buf_v.shape[0] // NEXP
    wid = lax.axis_index("s") * 2 + lax.axis_index("c")
    base = wid * (t_per_w * NEXP)
    pltpu.sync_copy(logits_hbm.at[pl.ds(base, t_per_w * NEXP)], buf_v)

    lane = lax.iota(jnp.int32, LANES)
    n_groups = t_per_w // LANES

    def group(g, _):
        idx_base = g * (LANES * NEXP) + lane * NEXP
        neg = jnp.full((LANES,), -jnp.inf, jnp.float32)
        m1 = neg
        m2 = neg
        i1 = jnp.zeros((LANES,), jnp.int32)
        i2 = jnp.zeros((LANES,), jnp.int32)
        z = jnp.zeros((LANES,), jnp.float32)
        for e in range(NEXP):
            l = plsc.load_gather(buf_v, [idx_base + e])
            z = z + jnp.exp(l)
            gt1 = l > m1
            gt2 = l > m2
            ev = jnp.full((LANES,), e, jnp.int32)
            m2 = jnp.where(gt1, m1, jnp.where(gt2, l, m2))
            i2 = jnp.where(gt1, i1, jnp.where(gt2, ev, i2))
            m1 = jnp.where(gt1, l, m1)
            i1 = jnp.where(gt1, ev, i1)
        p1 = jnp.clip(jnp.exp(m1) / z, EPS_, 1.0)
        p2 = jnp.clip(jnp.exp(m2) / z, EPS_, 1.0)
        s = jnp.maximum(p1 + p2, EPS_)
        o1 = p1 / s
        o2 = p2 / s
        pidx = g * (LANES * 2) + lane * 2
        plsc.store_scatter(probs_v, [pidx], o1)
        plsc.store_scatter(probs_v, [pidx + 1], o2)
        plsc.store_scatter(idx_v, [pidx], i1)
        plsc.store_scatter(idx_v, [pidx + 1], i2)
        return _

    lax.fori_loop(0, n_groups, group, 0, unroll=2)
    out_base = wid * (t_per_w * 2)
    pltpu.sync_copy(probs_v, probs_hbm.at[pl.ds(out_base, t_per_w * 2)])
    pltpu.sync_copy(idx_v, idx_hbm.at[pl.ds(out_base, t_per_w * 2)])


@jax.jit
def kernel(hidden_states, ln_weight, ln_bias, gate_weight):
    B, S, D = hidden_states.shape
    N = B * S
    x = hidden_states.reshape(N, D)
    w = ln_weight.reshape(1, D)
    b = ln_bias.reshape(1, D)
    gt = gate_weight.T
    E = gate_weight.shape[0]
    na = N // 4
    nb = N - na
    nblk_a = na // BLK
    nblk_b = nb // BLK

    def specs_for(off):
        return [
            pl.BlockSpec((BLK, D), lambda i: (i + off, 0)),
            pl.BlockSpec((1, D), lambda i: (0, 0)),
            pl.BlockSpec((1, D), lambda i: (0, 0)),
            pl.BlockSpec((D, E), lambda i: (0, 0)),
        ]

    logits_a = pl.pallas_call(
        _tc_logits_kernel,
        grid=(nblk_a,),
        in_specs=specs_for(0),
        out_specs=pl.BlockSpec((BLK, E), lambda i: (i, 0)),
        out_shape=jax.ShapeDtypeStruct((na, E), jnp.float32),
    )(x, w, b, gt)

    t_per_w = na // NWORK
    mesh = plsc.VectorSubcoreMesh(core_axis_name="c", subcore_axis_name="s")
    sc_topk = functools.partial(
        pl.kernel,
        mesh=mesh,
        compiler_params=pltpu.CompilerParams(needs_layout_passes=False),
        out_type=[
            jax.ShapeDtypeStruct((na * 2,), jnp.float32),
            jax.ShapeDtypeStruct((na * 2,), jnp.int32),
        ],
        scratch_types=[
            pltpu.VMEM((t_per_w * NEXP,), jnp.float32),
            pltpu.VMEM((t_per_w * 2,), jnp.float32),
            pltpu.VMEM((t_per_w * 2,), jnp.int32),
        ],
    )(_sc_topk_body)
    probs_a, idx_a = sc_topk(logits_a.reshape(-1))

    logits_b, probs_b, idx_b = pl.pallas_call(
        _tc_full_kernel,
        grid=(nblk_b,),
        in_specs=specs_for(nblk_a),
        out_specs=[
            pl.BlockSpec((BLK, E), lambda i: (i, 0)),
            pl.BlockSpec((BLK, 2), lambda i: (i, 0)),
            pl.BlockSpec((BLK, 2), lambda i: (i, 0)),
        ],
        out_shape=[
            jax.ShapeDtypeStruct((nb, E), jnp.float32),
            jax.ShapeDtypeStruct((nb, 2), jnp.float32),
            jax.ShapeDtypeStruct((nb, 2), jnp.int32),
        ],
    )(x, w, b, gt)

    probs = jnp.concatenate([probs_a.reshape(na, 2), probs_b], axis=0)
    idx = jnp.concatenate([idx_a.reshape(na, 2), idx_b], axis=0)
    logits = jnp.concatenate([logits_a, logits_b], axis=0)
    return probs, idx, logits


# speedup vs baseline: 1.7252x; 1.0038x over previous
import functools

import jax
import jax.numpy as jnp
from jax import lax
from jax.experimental import pallas as pl
from jax.experimental.pallas import tpu as pltpu
from jax.experimental.pallas import tpu_sc as plsc

EPS_ = 1e-4
BLK = 2048
NEXP = 64
LANES = 16
NWORK = 32


def _ln_logits(x, w, b, gt):
    mean = jnp.mean(x, axis=1, keepdims=True)
    xc = x - mean
    var = jnp.mean(xc * xc, axis=1, keepdims=True)
    hn = xc * lax.rsqrt(var + 1e-5) * w + b
    hn = jnp.clip(hn, -50.0, 50.0)
    logits = jax.lax.dot_general(
        hn, gt, (((1,), (0,)), ((), ())),
        preferred_element_type=jnp.float32,
    )
    return jnp.clip(logits, -10.0, 10.0)


def _tc_logits_kernel(x_ref, w_ref, b_ref, gt_ref, logits_ref):
    logits_ref[...] = _ln_logits(x_ref[...], w_ref[...], b_ref[...], gt_ref[...])


def _tc_full_kernel(x_ref, w_ref, b_ref, gt_ref, logits_ref, probs_ref, idx_ref):
    logits = _ln_logits(x_ref[...], w_ref[...], b_ref[...], gt_ref[...])
    logits_ref[...] = logits
    m = jnp.max(logits, axis=1, keepdims=True)
    e = jnp.exp(logits - m)
    z = jnp.sum(e, axis=1, keepdims=True)
    p = jnp.clip(e / z, EPS_, 1.0)
    iota = lax.broadcasted_iota(jnp.int32, p.shape, 1)
    m1 = jnp.max(p, axis=1, keepdims=True)
    i1 = jnp.min(jnp.where(p == m1, iota, NEXP), axis=1, keepdims=True)
    pm = jnp.where(iota == i1, -1.0, p)
    m2 = jnp.max(pm, axis=1, keepdims=True)
    i2 = jnp.min(jnp.where(pm == m2, iota, NEXP), axis=1, keepdims=True)
    s = jnp.maximum(m1 + m2, EPS_)
    probs_ref[...] = jnp.concatenate([m1 / s, m2 / s], axis=1)
    idx_ref[...] = jnp.concatenate([i1, i2], axis=1)


def _sc_topk_body(logits_hbm, probs_hbm, idx_hbm, buf_v, probs_v, idx_v):
    t_per_w = buf_v.shape[0] // NEXP
    wid = lax.axis_index("s") * 2 + lax.axis_index("c")
    base = wid * (t_per_w * NEXP)
    pltpu.sync_copy(logits_hbm.at[pl.ds(base, t_per_w * NEXP)], buf_v)

    lane = lax.iota(jnp.int32, LANES)
    n_groups = t_per_w // LANES

    def group(g, _):
        idx_base = g * (LANES * NEXP) + lane * NEXP
        neg = jnp.full((LANES,), -jnp.inf, jnp.float32)
        m1 = neg
        m2 = neg
        i1 = jnp.zeros((LANES,), jnp.int32)
        i2 = jnp.zeros((LANES,), jnp.int32)
        z = jnp.zeros((LANES,), jnp.float32)
        for e in range(NEXP):
            l = plsc.load_gather(buf_v, [idx_base + e])
            z = z + jnp.exp(l)
            gt1 = l > m1
            gt2 = l > m2
            ev = jnp.full((LANES,), e, jnp.int32)
            m2 = jnp.where(gt1, m1, jnp.where(gt2, l, m2))
            i2 = jnp.where(gt1, i1, jnp.where(gt2, ev, i2))
            m1 = jnp.where(gt1, l, m1)
            i1 = jnp.where(gt1, ev, i1)
        p1 = jnp.clip(jnp.exp(m1) / z, EPS_, 1.0)
        p2 = jnp.clip(jnp.exp(m2) / z, EPS_, 1.0)
        s = jnp.maximum(p1 + p2, EPS_)
        o1 = p1 / s
        o2 = p2 / s
        pidx = g * (LANES * 2) + lane * 2
        plsc.store_scatter(probs_v, [pidx], o1)
        plsc.store_scatter(probs_v, [pidx + 1], o2)
        plsc.store_scatter(idx_v, [pidx], i1)
        plsc.store_scatter(idx_v, [pidx + 1], i2)
        return _

    lax.fori_loop(0, n_groups, group, 0, unroll=2)
    out_base = wid * (t_per_w * 2)
    pltpu.sync_copy(probs_v, probs_hbm.at[pl.ds(out_base, t_per_w * 2)])
    pltpu.sync_copy(idx_v, idx_hbm.at[pl.ds(out_base, t_per_w * 2)])


@jax.jit
def kernel(hidden_states, ln_weight, ln_bias, gate_weight):
    B, S, D = hidden_states.shape
    N = B * S
    x = hidden_states.reshape(N, D)
    w = ln_weight.reshape(1, D)
    b = ln_bias.reshape(1, D)
    gt = gate_weight.T
    E = gate_weight.shape[0]
    na = N // 8
    nb = N - na
    nblk_a = na // BLK
    nblk_b = nb // BLK

    def specs_for(off):
        return [
            pl.BlockSpec((BLK, D), lambda i: (i + off, 0)),
            pl.BlockSpec((1, D), lambda i: (0, 0)),
            pl.BlockSpec((1, D), lambda i: (0, 0)),
            pl.BlockSpec((D, E), lambda i: (0, 0)),
        ]

    logits_a = pl.pallas_call(
        _tc_logits_kernel,
        grid=(nblk_a,),
        in_specs=specs_for(0),
        out_specs=pl.BlockSpec((BLK, E), lambda i: (i, 0)),
        out_shape=jax.ShapeDtypeStruct((na, E), jnp.float32),
    )(x, w, b, gt)

    t_per_w = na // NWORK
    mesh = plsc.VectorSubcoreMesh(core_axis_name="c", subcore_axis_name="s")
    sc_topk = functools.partial(
        pl.kernel,
        mesh=mesh,
        compiler_params=pltpu.CompilerParams(needs_layout_passes=False),
        out_type=[
            jax.ShapeDtypeStruct((na * 2,), jnp.float32),
            jax.ShapeDtypeStruct((na * 2,), jnp.int32),
        ],
        scratch_types=[
            pltpu.VMEM((t_per_w * NEXP,), jnp.float32),
            pltpu.VMEM((t_per_w * 2,), jnp.float32),
            pltpu.VMEM((t_per_w * 2,), jnp.int32),
        ],
    )(_sc_topk_body)
    probs_a, idx_a = sc_topk(logits_a.reshape(-1))

    logits_b, probs_b, idx_b = pl.pallas_call(
        _tc_full_kernel,
        grid=(nblk_b,),
        in_specs=specs_for(nblk_a),
        out_specs=[
            pl.BlockSpec((BLK, E), lambda i: (i, 0)),
            pl.BlockSpec((BLK, 2), lambda i: (i, 0)),
            pl.BlockSpec((BLK, 2), lambda i: (i, 0)),
        ],
        out_shape=[
            jax.ShapeDtypeStruct((nb, E), jnp.float32),
            jax.ShapeDtypeStruct((nb, 2), jnp.float32),
            jax.ShapeDtypeStruct((nb, 2), jnp.int32),
        ],
    )(x, w, b, gt)

    probs = jnp.concatenate([probs_a.reshape(na, 2), probs_b], axis=0)
    idx = jnp.concatenate([idx_a.reshape(na, 2), idx_b], axis=0)
    logits = jnp.concatenate([logits_a, logits_b], axis=0)
    return probs, idx, logits


# speedup vs baseline: 1.7648x; 1.0230x over previous
import functools

import jax
import jax.numpy as jnp
from jax import lax
from jax.experimental import pallas as pl
from jax.experimental.pallas import tpu as pltpu
from jax.experimental.pallas import tpu_sc as plsc

EPS_ = 1e-4
BLK = 2048
NEXP = 64
LANES = 16
NWORK = 32


def _ln_logits(x, w, b, gt):
    mean = jnp.mean(x, axis=1, keepdims=True)
    xc = x - mean
    var = jnp.mean(xc * xc, axis=1, keepdims=True)
    hn = xc * lax.rsqrt(var + 1e-5) * w + b
    hn = jnp.clip(hn, -50.0, 50.0)
    logits = jax.lax.dot_general(
        hn, gt, (((1,), (0,)), ((), ())),
        preferred_element_type=jnp.float32,
    )
    return jnp.clip(logits, -10.0, 10.0)


def _tc_logits_kernel(x_ref, w_ref, b_ref, gt_ref, logits_ref):
    logits_ref[...] = _ln_logits(x_ref[...], w_ref[...], b_ref[...], gt_ref[...])


def _tc_full_kernel(x_ref, w_ref, b_ref, gt_ref, logits_ref, probs_ref, idx_ref):
    logits = _ln_logits(x_ref[...], w_ref[...], b_ref[...], gt_ref[...])
    logits_ref[...] = logits
    m = jnp.max(logits, axis=1, keepdims=True)
    e = jnp.exp(logits - m)
    z = jnp.sum(e, axis=1, keepdims=True)
    p = jnp.clip(e / z, EPS_, 1.0)
    iota = lax.broadcasted_iota(jnp.int32, p.shape, 1)
    m1 = jnp.max(p, axis=1, keepdims=True)
    i1 = jnp.min(jnp.where(p == m1, iota, NEXP), axis=1, keepdims=True)
    pm = jnp.where(iota == i1, -1.0, p)
    m2 = jnp.max(pm, axis=1, keepdims=True)
    i2 = jnp.min(jnp.where(pm == m2, iota, NEXP), axis=1, keepdims=True)
    s = jnp.maximum(m1 + m2, EPS_)
    probs_ref[...] = jnp.concatenate([m1 / s, m2 / s], axis=1)
    idx_ref[...] = jnp.concatenate([i1, i2], axis=1)


def _sc_topk_body(logits_hbm, probs_hbm, idx_hbm, buf_v, probs_v, idx_v):
    t_per_w = buf_v.shape[0] // NEXP
    wid = lax.axis_index("s") * 2 + lax.axis_index("c")
    base = wid * (t_per_w * NEXP)
    pltpu.sync_copy(logits_hbm.at[pl.ds(base, t_per_w * NEXP)], buf_v)

    lane = lax.iota(jnp.int32, LANES)
    n_groups = t_per_w // LANES

    def group(g, _):
        idx_base = g * (LANES * NEXP) + lane * NEXP
        neg = jnp.full((LANES,), -jnp.inf, jnp.float32)
        m1 = neg
        m2 = neg
        i1 = jnp.zeros((LANES,), jnp.int32)
        i2 = jnp.zeros((LANES,), jnp.int32)
        z = jnp.zeros((LANES,), jnp.float32)
        for e in range(NEXP):
            l = plsc.load_gather(buf_v, [idx_base + e])
            z = z + jnp.exp(l)
            gt1 = l > m1
            gt2 = l > m2
            ev = jnp.full((LANES,), e, jnp.int32)
            m2 = jnp.where(gt1, m1, jnp.where(gt2, l, m2))
            i2 = jnp.where(gt1, i1, jnp.where(gt2, ev, i2))
            m1 = jnp.where(gt1, l, m1)
            i1 = jnp.where(gt1, ev, i1)
        p1 = jnp.clip(jnp.exp(m1) / z, EPS_, 1.0)
        p2 = jnp.clip(jnp.exp(m2) / z, EPS_, 1.0)
        s = jnp.maximum(p1 + p2, EPS_)
        o1 = p1 / s
        o2 = p2 / s
        pidx = g * (LANES * 2) + lane * 2
        plsc.store_scatter(probs_v, [pidx], o1)
        plsc.store_scatter(probs_v, [pidx + 1], o2)
        plsc.store_scatter(idx_v, [pidx], i1)
        plsc.store_scatter(idx_v, [pidx + 1], i2)
        return _

    lax.fori_loop(0, n_groups, group, 0, unroll=2)
    out_base = wid * (t_per_w * 2)
    pltpu.sync_copy(probs_v, probs_hbm.at[pl.ds(out_base, t_per_w * 2)])
    pltpu.sync_copy(idx_v, idx_hbm.at[pl.ds(out_base, t_per_w * 2)])


@jax.jit
def kernel(hidden_states, ln_weight, ln_bias, gate_weight):
    B, S, D = hidden_states.shape
    N = B * S
    x = hidden_states.reshape(N, D)
    w = ln_weight.reshape(1, D)
    b = ln_bias.reshape(1, D)
    gt = gate_weight.T
    E = gate_weight.shape[0]
    na = N // 4
    nb = N - na
    nblk_a = na // BLK
    nblk_b = nb // BLK

    def specs_for(off):
        return [
            pl.BlockSpec((BLK, D), lambda i: (i + off, 0)),
            pl.BlockSpec((1, D), lambda i: (0, 0)),
            pl.BlockSpec((1, D), lambda i: (0, 0)),
            pl.BlockSpec((D, E), lambda i: (0, 0)),
        ]

    logits_a = pl.pallas_call(
        _tc_logits_kernel,
        grid=(nblk_a,),
        in_specs=specs_for(0),
        out_specs=pl.BlockSpec((BLK, E), lambda i: (i, 0)),
        out_shape=jax.ShapeDtypeStruct((na, E), jnp.float32),
    )(x, w, b, gt)

    t_per_w = na // NWORK
    mesh = plsc.VectorSubcoreMesh(core_axis_name="c", subcore_axis_name="s")
    sc_topk = functools.partial(
        pl.kernel,
        mesh=mesh,
        compiler_params=pltpu.CompilerParams(needs_layout_passes=False),
        out_type=[
            jax.ShapeDtypeStruct((na * 2,), jnp.float32),
            jax.ShapeDtypeStruct((na * 2,), jnp.int32),
        ],
        scratch_types=[
            pltpu.VMEM((t_per_w * NEXP,), jnp.float32),
            pltpu.VMEM((t_per_w * 2,), jnp.float32),
            pltpu.VMEM((t_per_w * 2,), jnp.int32),
        ],
    )(_sc_topk_body)
    probs_a, idx_a = sc_topk(logits_a.reshape(-1))

    logits_b, probs_b, idx_b = pl.pallas_call(
        _tc_full_kernel,
        grid=(nblk_b,),
        in_specs=specs_for(nblk_a),
        out_specs=[
            pl.BlockSpec((BLK, E), lambda i: (i, 0)),
            pl.BlockSpec((BLK, 2), lambda i: (i, 0)),
            pl.BlockSpec((BLK, 2), lambda i: (i, 0)),
        ],
        out_shape=[
            jax.ShapeDtypeStruct((nb, E), jnp.float32),
            jax.ShapeDtypeStruct((nb, 2), jnp.float32),
            jax.ShapeDtypeStruct((nb, 2), jnp.int32),
        ],
    )(x, w, b, gt)

    probs = jnp.concatenate([probs_a.reshape(na, 2), probs_b], axis=0)
    idx = jnp.concatenate([idx_a.reshape(na, 2), idx_b], axis=0)
    logits = jnp.concatenate([logits_a, logits_b], axis=0)
    return probs, idx, logits
